# baseline (device time: 9511 ns/iter reference)
import os

import jax
import jax.numpy as jnp
from jax import lax
from jax.experimental import pallas as pl
from jax.experimental.pallas import tpu as pltpu

N_DEV = 8
E_PER = 2
MODE = os.environ.get("DIAG_MODE", "compute")


def kernel(x, router_W, route_idx, expert_W):
    del router_W
    n, d = x.shape
    h = expert_W.shape[-1]

    def body(x_ref, idx_ref, w_ref, out_ref):
        me = lax.axis_index("i")
        if MODE != "local":
            barrier_sem = pltpu.get_barrier_semaphore()
            for p in range(N_DEV):
                @pl.when(me != p)
                def _(p=p):
                    pl.semaphore_signal(
                        barrier_sem, inc=1,
                        device_id=(p,), device_id_type=pl.DeviceIdType.MESH,
                    )
            pl.semaphore_wait(barrier_sem, N_DEV - 1)

        if MODE == "barrier":
            out_ref[:, :] = jnp.zeros((n, h), jnp.float32)
            return

        e0 = me * E_PER
        wcat = w_ref[:, :, :].astype(jnp.bfloat16).reshape(E_PER * d, h)
        xm0 = jnp.where(idx_ref[:, :] == e0, x_ref[:, :], 0.0)
        xm1 = jnp.where(idx_ref[:, :] == e0 + 1, x_ref[:, :], 0.0)
        xcat = jnp.concatenate([xm0, xm1], axis=1).astype(jnp.bfloat16)
        out_ref[:, :] = jnp.dot(xcat, wcat, preferred_element_type=jnp.float32)

    return pl.pallas_call(
        body,
        out_shape=jax.ShapeDtypeStruct((n, h), jnp.float32),
        in_specs=[pl.BlockSpec(memory_space=pltpu.VMEM)] * 3,
        out_specs=pl.BlockSpec(memory_space=pltpu.VMEM),
        compiler_params=(
            None if MODE == "local" else pltpu.CompilerParams(collective_id=0)
        ),
    )(x, route_idx, expert_W)
